# baseline (speedup 1.0000x reference)
import functools

import jax
import jax.numpy as jnp
from jax import lax
from jax.experimental import pallas as pl
from jax.experimental.pallas import tpu as pltpu
from jax.experimental.pallas import tpu_sc as plsc

_NUM_OBJS = 100000
_NUM_COLORS = 256
_N = _NUM_OBJS * _NUM_COLORS

_NUM_TILES = 32
_PER_TILE = _N // _NUM_TILES
_CHUNK = 6_400
_NBUF = 5
_VECS = _CHUNK // 16
_UNROLL = 4
_NCHUNK = _PER_TILE // _CHUNK
_BIG_I32 = 2**31 - 1

_MESH = plsc.VectorSubcoreMesh(core_axis_name="c", subcore_axis_name="s")


@functools.partial(
    pl.kernel,
    mesh=_MESH,
    out_type=(
        jax.ShapeDtypeStruct((_NUM_TILES, 16), jnp.float32),
        jax.ShapeDtypeStruct((_NUM_TILES, 16), jnp.int32),
    ),
    scratch_types=(
        [pltpu.VMEM((_CHUNK,), jnp.float32) for _ in range(2 * _NBUF)]
        + [
            pltpu.VMEM((16,), jnp.float32),
            pltpu.VMEM((16,), jnp.int32),
            pltpu.SemaphoreType.DMA((_NBUF,)),
            pltpu.SemaphoreType.DMA((_NBUF,)),
        ]
    ),
    compiler_params=pltpu.CompilerParams(needs_layout_passes=False),
)
def _scan(x_hbm, m_hbm, pmax_hbm, pidx_hbm, *_scr):
    xb = list(_scr[:_NBUF])
    mb = list(_scr[_NBUF:2 * _NBUF])
    stage_f, stage_i, sx, sm = _scr[2 * _NBUF:]
    wid = lax.axis_index("c") * 16 + lax.axis_index("s")
    base = wid * _PER_TILE
    iota = lax.iota(jnp.int32, 16)

    def start_chunk(g, b):
        off = base + g * _CHUNK
        pltpu.async_copy(x_hbm.at[pl.ds(off, _CHUNK)], xb[b], sx.at[b])
        pltpu.async_copy(m_hbm.at[pl.ds(off, _CHUNK)], mb[b], sm.at[b])

    def wait_chunk(g, b):
        off = base + g * _CHUNK
        pltpu.make_async_copy(x_hbm.at[pl.ds(off, _CHUNK)], xb[b], sx.at[b]).wait()
        pltpu.make_async_copy(m_hbm.at[pl.ds(off, _CHUNK)], mb[b], sm.at[b]).wait()

    for b in range(_NBUF - 1):
        start_chunk(b, b)

    ninf = jnp.full((16,), -jnp.inf, jnp.float32)
    zero_i = jnp.zeros((16,), jnp.int32)
    init = tuple([ninf] * _UNROLL) + tuple([zero_i] * _UNROLL)

    def consume(g, b, carry, refill):
        if refill:
            @pl.when(g + _NBUF - 1 < _NCHUNK)
            def _():
                start_chunk(g + _NBUF - 1, (b + _NBUF - 1) % _NBUF)
        wait_chunk(g, b)

        gbase = base + g * _CHUNK

        def inner(j, c):
            vms = list(c[:_UNROLL])
            vis = list(c[_UNROLL:])
            jb = j * (16 * _UNROLL)
            ivec = gbase + jb + iota
            for u in range(_UNROLL):
                o = jb + u * 16
                v = xb[b][pl.ds(o, 16)] * mb[b][pl.ds(o, 16)]
                cur = ivec + (u * 16)
                p = v > vms[u]
                vms[u] = jnp.where(p, v, vms[u])
                vis[u] = jnp.where(p, cur, vis[u])
            return tuple(vms) + tuple(vis)

        return lax.fori_loop(0, _VECS // _UNROLL, inner, carry)

    def outer(g2, carry):
        for b in range(_NBUF):
            carry = consume(g2 * _NBUF + b, b, carry, refill=True)
        return carry

    acc = lax.fori_loop(0, _NCHUNK // _NBUF, outer, init)
    _G0 = (_NCHUNK // _NBUF) * _NBUF
    for _r in range(_NCHUNK % _NBUF):
        acc = consume(_G0 + _r, _r, acc, refill=(_G0 + _r + _NBUF - 1 < _NCHUNK))

    vm, vi = acc[0], acc[_UNROLL]
    for a in range(1, _UNROLL):
        v2, i2 = acc[a], acc[_UNROLL + a]
        p = (v2 > vm) | ((v2 == vm) & (i2 < vi))
        vm = jnp.where(p, v2, vm)
        vi = jnp.where(p, i2, vi)
    mx = jnp.max(vm)
    gidx = jnp.min(jnp.where(vm == mx, vi, _BIG_I32))

    stage_f[...] = jnp.full((16,), mx, jnp.float32)
    stage_i[...] = jnp.full((16,), gidx, jnp.int32)
    pltpu.sync_copy(stage_f, pmax_hbm.at[wid])
    pltpu.sync_copy(stage_i, pidx_hbm.at[wid])


def _merge_body(x_hbm, pm_ref, pi_ref, out_ref, child, sem):
    pm = pm_ref[...]
    pi = pi_ref[...]
    mx = jnp.max(pm)
    gidx = jnp.min(jnp.where(pm == mx, pi, _BIG_I32))

    child_obj = ((gidx // _NUM_COLORS) + 1) % _NUM_OBJS
    copy = pltpu.make_async_copy(
        x_hbm.at[pl.ds(child_obj * _NUM_COLORS, _NUM_COLORS)], child, sem
    )
    copy.start()
    copy.wait()

    cv = child[...].reshape(1, _NUM_COLORS)
    iota_l = lax.broadcasted_iota(jnp.int32, (1, _NUM_COLORS), 1)
    cmx = jnp.max(cv)
    cidx = jnp.min(jnp.where(cv == cmx, iota_l, _BIG_I32))
    cnew = (cidx + 1) % _NUM_COLORS
    out_ref[...] = (iota_l == cnew).astype(jnp.float32)


def _merge(x1d, pmax, pidx):
    return pl.pallas_call(
        _merge_body,
        out_shape=jax.ShapeDtypeStruct((1, _NUM_COLORS), jnp.float32),
        in_specs=[
            pl.BlockSpec(memory_space=pltpu.HBM),
            pl.BlockSpec(memory_space=pltpu.VMEM),
            pl.BlockSpec(memory_space=pltpu.VMEM),
        ],
        out_specs=pl.BlockSpec(memory_space=pltpu.VMEM),
        scratch_shapes=[
            pltpu.VMEM((_NUM_COLORS,), jnp.float32),
            pltpu.SemaphoreType.DMA,
        ],
    )(x1d, pmax, pidx)


def kernel(x, mask):
    xf = x.reshape(_N)
    mf = mask.reshape(_N)
    pmax, pidx = _scan(xf, mf)
    out = _merge(xf, pmax, pidx)
    return out.reshape(_NUM_COLORS)


# speedup vs baseline: 1.0046x; 1.0046x over previous
import functools

import jax
import jax.numpy as jnp
from jax import lax
from jax.experimental import pallas as pl
from jax.experimental.pallas import tpu as pltpu
from jax.experimental.pallas import tpu_sc as plsc

_NUM_OBJS = 100000
_NUM_COLORS = 256
_N = _NUM_OBJS * _NUM_COLORS

_NUM_TILES = 32
_PER_TILE = _N // _NUM_TILES
_CHUNK = 6_400
_NBUF = 5
_VECS = _CHUNK // 16
_UNROLL = 8
_NCHUNK = _PER_TILE // _CHUNK
_BIG_I32 = 2**31 - 1

_MESH = plsc.VectorSubcoreMesh(core_axis_name="c", subcore_axis_name="s")


@functools.partial(
    pl.kernel,
    mesh=_MESH,
    out_type=(
        jax.ShapeDtypeStruct((_NUM_TILES, 16), jnp.float32),
        jax.ShapeDtypeStruct((_NUM_TILES, 16), jnp.int32),
    ),
    scratch_types=(
        [pltpu.VMEM((_CHUNK,), jnp.float32) for _ in range(2 * _NBUF)]
        + [
            pltpu.VMEM((16,), jnp.float32),
            pltpu.VMEM((16,), jnp.int32),
            pltpu.SemaphoreType.DMA((_NBUF,)),
            pltpu.SemaphoreType.DMA((_NBUF,)),
        ]
    ),
    compiler_params=pltpu.CompilerParams(needs_layout_passes=False),
)
def _scan(x_hbm, m_hbm, pmax_hbm, pidx_hbm, *_scr):
    xb = list(_scr[:_NBUF])
    mb = list(_scr[_NBUF:2 * _NBUF])
    stage_f, stage_i, sx, sm = _scr[2 * _NBUF:]
    wid = lax.axis_index("c") * 16 + lax.axis_index("s")
    iota = lax.iota(jnp.int32, 16)

    def chunk_off(g):
        return (g * _NUM_TILES + wid) * _CHUNK

    def start_chunk(g, b):
        off = chunk_off(g)
        pltpu.async_copy(x_hbm.at[pl.ds(off, _CHUNK)], xb[b], sx.at[b])
        pltpu.async_copy(m_hbm.at[pl.ds(off, _CHUNK)], mb[b], sm.at[b])

    def wait_chunk(g, b):
        off = chunk_off(g)
        pltpu.make_async_copy(x_hbm.at[pl.ds(off, _CHUNK)], xb[b], sx.at[b]).wait()
        pltpu.make_async_copy(m_hbm.at[pl.ds(off, _CHUNK)], mb[b], sm.at[b]).wait()

    for b in range(_NBUF - 1):
        start_chunk(b, b)

    ninf = jnp.full((16,), -jnp.inf, jnp.float32)
    zero_i = jnp.zeros((16,), jnp.int32)
    init = tuple([ninf] * _UNROLL) + tuple([zero_i] * _UNROLL)

    def consume(g, b, carry, refill):
        if refill:
            @pl.when(g + _NBUF - 1 < _NCHUNK)
            def _():
                start_chunk(g + _NBUF - 1, (b + _NBUF - 1) % _NBUF)
        wait_chunk(g, b)

        gbase = chunk_off(g)

        def inner(j, c):
            vms = list(c[:_UNROLL])
            vis = list(c[_UNROLL:])
            jb = j * (16 * _UNROLL)
            ivec = gbase + jb + iota
            for u in range(_UNROLL):
                o = jb + u * 16
                v = xb[b][pl.ds(o, 16)] * mb[b][pl.ds(o, 16)]
                cur = ivec + (u * 16)
                p = v > vms[u]
                vms[u] = jnp.where(p, v, vms[u])
                vis[u] = jnp.where(p, cur, vis[u])
            return tuple(vms) + tuple(vis)

        return lax.fori_loop(0, _VECS // _UNROLL, inner, carry)

    def outer(g2, carry):
        for b in range(_NBUF):
            carry = consume(g2 * _NBUF + b, b, carry, refill=True)
        return carry

    acc = lax.fori_loop(0, _NCHUNK // _NBUF, outer, init)
    _G0 = (_NCHUNK // _NBUF) * _NBUF
    for _r in range(_NCHUNK % _NBUF):
        acc = consume(_G0 + _r, _r, acc, refill=(_G0 + _r + _NBUF - 1 < _NCHUNK))

    vm, vi = acc[0], acc[_UNROLL]
    for a in range(1, _UNROLL):
        v2, i2 = acc[a], acc[_UNROLL + a]
        p = (v2 > vm) | ((v2 == vm) & (i2 < vi))
        vm = jnp.where(p, v2, vm)
        vi = jnp.where(p, i2, vi)
    mx = jnp.max(vm)
    gidx = jnp.min(jnp.where(vm == mx, vi, _BIG_I32))

    stage_f[...] = jnp.full((16,), mx, jnp.float32)
    stage_i[...] = jnp.full((16,), gidx, jnp.int32)
    pltpu.sync_copy(stage_f, pmax_hbm.at[wid])
    pltpu.sync_copy(stage_i, pidx_hbm.at[wid])


def _merge_body(x_hbm, pm_ref, pi_ref, out_ref, child, sem):
    pm = pm_ref[...]
    pi = pi_ref[...]
    mx = jnp.max(pm)
    gidx = jnp.min(jnp.where(pm == mx, pi, _BIG_I32))

    child_obj = ((gidx // _NUM_COLORS) + 1) % _NUM_OBJS
    copy = pltpu.make_async_copy(
        x_hbm.at[pl.ds(child_obj * _NUM_COLORS, _NUM_COLORS)], child, sem
    )
    copy.start()
    copy.wait()

    cv = child[...].reshape(1, _NUM_COLORS)
    iota_l = lax.broadcasted_iota(jnp.int32, (1, _NUM_COLORS), 1)
    cmx = jnp.max(cv)
    cidx = jnp.min(jnp.where(cv == cmx, iota_l, _BIG_I32))
    cnew = (cidx + 1) % _NUM_COLORS
    out_ref[...] = (iota_l == cnew).astype(jnp.float32)


def _merge(x1d, pmax, pidx):
    return pl.pallas_call(
        _merge_body,
        out_shape=jax.ShapeDtypeStruct((1, _NUM_COLORS), jnp.float32),
        in_specs=[
            pl.BlockSpec(memory_space=pltpu.HBM),
            pl.BlockSpec(memory_space=pltpu.VMEM),
            pl.BlockSpec(memory_space=pltpu.VMEM),
        ],
        out_specs=pl.BlockSpec(memory_space=pltpu.VMEM),
        scratch_shapes=[
            pltpu.VMEM((_NUM_COLORS,), jnp.float32),
            pltpu.SemaphoreType.DMA,
        ],
    )(x1d, pmax, pidx)


def kernel(x, mask):
    xf = x.reshape(_N)
    mf = mask.reshape(_N)
    pmax, pidx = _scan(xf, mf)
    out = _merge(xf, pmax, pidx)
    return out.reshape(_NUM_COLORS)
